# baseline (speedup 1.0000x reference)
import functools

import jax
import jax.numpy as jnp
from jax import lax
from jax.experimental import pallas as pl
from jax.experimental.pallas import tpu as pltpu
from jax.experimental.pallas import tpu_sc as plsc

N = 10000
E = 320000
D = 128
ED = 16

NC = 2
NS = 16
L = 16
NW = NC * NS
EPW = E // NW
K = 40
NCHUNK = EPW // K
WROWS = NCHUNK
NPAD = 10240
RPT = NPAD // NS

NBLK = 1000
EBLK = 4000


def _mlp2_body(x_ref, w1_ref, b1_ref, w2_ref, b2_ref, o_ref):
    h = jnp.maximum(
        jnp.dot(x_ref[...], w1_ref[...], preferred_element_type=jnp.float32)
        + b1_ref[...], 0.0)
    o_ref[...] = (
        jnp.dot(h, w2_ref[...], preferred_element_type=jnp.float32)
        + b2_ref[...])


def _mlp2(x, w1, b1, w2, b2, dout):
    n = x.shape[0]
    return pl.pallas_call(
        _mlp2_body,
        grid=(n // NBLK,),
        in_specs=[
            pl.BlockSpec((NBLK, x.shape[1]), lambda i: (i, 0)),
            pl.BlockSpec(w1.shape, lambda i: (0, 0)),
            pl.BlockSpec((1, D), lambda i: (0, 0)),
            pl.BlockSpec(w2.shape, lambda i: (0, 0)),
            pl.BlockSpec((1, dout), lambda i: (0, 0)),
        ],
        out_specs=pl.BlockSpec((NBLK, dout), lambda i: (i, 0)),
        out_shape=jax.ShapeDtypeStruct((n, dout), jnp.float32),
    )(x, w1, b1.reshape(1, D), w2, b2.reshape(1, dout))


def _ab_body(x_ref, wa_ref, wb_ref, mb_ref, a_ref, b_ref):
    xv = x_ref[...]
    a_ref[...] = jnp.dot(xv, wa_ref[...], preferred_element_type=jnp.float32)
    b_ref[...] = (
        jnp.dot(xv, wb_ref[...], preferred_element_type=jnp.float32)
        + mb_ref[...])


def _ab(x, wa, wb, mb):
    return pl.pallas_call(
        _ab_body,
        grid=(N // NBLK,),
        in_specs=[
            pl.BlockSpec((NBLK, D), lambda i: (i, 0)),
            pl.BlockSpec((D, D), lambda i: (0, 0)),
            pl.BlockSpec((D, D), lambda i: (0, 0)),
            pl.BlockSpec((1, D), lambda i: (0, 0)),
        ],
        out_specs=[
            pl.BlockSpec((NBLK, D), lambda i: (i, 0)),
            pl.BlockSpec((NBLK, D), lambda i: (i, 0)),
        ],
        out_shape=[
            jax.ShapeDtypeStruct((N, D), jnp.float32),
            jax.ShapeDtypeStruct((N, D), jnp.float32),
        ],
    )(x, wa, wb, mb.reshape(1, D))


def _ep_body(ef_ref, we_ref, o_ref):
    o_ref[...] = jnp.dot(
        ef_ref[...], we_ref[...], preferred_element_type=jnp.float32
    ).astype(jnp.bfloat16)


def _ep(ef, we):
    return pl.pallas_call(
        _ep_body,
        grid=(E // EBLK,),
        in_specs=[
            pl.BlockSpec((EBLK, ED), lambda i: (i, 0)),
            pl.BlockSpec((ED, D), lambda i: (0, 0)),
        ],
        out_specs=pl.BlockSpec((EBLK, D), lambda i: (i, 0)),
        out_shape=jax.ShapeDtypeStruct((E, D), jnp.bfloat16),
    )(ef, we)


def _upd_body(x_ref, g0_ref, g1_ref, wx_ref, wa_ref, ub_ref, o_ref):
    xv = x_ref[...]
    agg = g0_ref[0] + g1_ref[0]
    o_ref[...] = xv + jnp.maximum(
        jnp.dot(xv, wx_ref[...], preferred_element_type=jnp.float32)
        + jnp.dot(agg, wa_ref[...], preferred_element_type=jnp.float32)
        + ub_ref[...], 0.0)


def _upd(x, aggs, wx, wa, ub):
    return pl.pallas_call(
        _upd_body,
        grid=(N // NBLK,),
        in_specs=[
            pl.BlockSpec((NBLK, D), lambda i: (i, 0)),
            pl.BlockSpec((1, NBLK, D), lambda i: (0, i, 0)),
            pl.BlockSpec((1, NBLK, D), lambda i: (1, i, 0)),
            pl.BlockSpec((D, D), lambda i: (0, 0)),
            pl.BlockSpec((D, D), lambda i: (0, 0)),
            pl.BlockSpec((1, D), lambda i: (0, 0)),
        ],
        out_specs=pl.BlockSpec((NBLK, D), lambda i: (i, 0)),
        out_shape=jax.ShapeDtypeStruct((N, D), jnp.float32),
    )(x, aggs, aggs, wx, wa, ub.reshape(1, D))


def _sc_edge_body(a_hbm, b_hbm, ep_hbm, sd_hbm, out_hbm,
                  sdidx, abuf, bbuf, epbuf, mbuf, agg, isem, asem, bsem,
                  esem):
    cid = lax.axis_index("c")
    sid = lax.axis_index("s")
    wid = sid * NC + cid
    wrow = wid * WROWS
    ebase = wid * EPW

    def zrow(r, carry):
        for c in range(D // L):
            mbuf[r, pl.ds(c * L, L)] = jnp.zeros((L,), jnp.float32)
        return carry
    lax.fori_loop(0, K, zrow, 0)

    def zcopy(p, carry):
        off = pl.multiple_of(sid * RPT + p * K, K)
        pltpu.sync_copy(mbuf, agg.at[pl.ds(off, K)])
        return carry
    lax.fori_loop(0, RPT // K, zcopy, 0)
    plsc.subcore_barrier()

    def issue_gathers(c, s):
        pltpu.async_copy(a_hbm.at[sdidx[s].at[0]], abuf[s], asem[s])
        pltpu.async_copy(b_hbm.at[sdidx[s].at[1]], bbuf[s], bsem[s])
        pltpu.async_copy(ep_hbm.at[pl.ds(ebase + c * K, K)], epbuf[s],
                         esem[s])

    def wait_gathers(s):
        pltpu.make_async_copy(a_hbm.at[sdidx[s].at[0]], abuf[s],
                              asem[s]).wait()
        pltpu.make_async_copy(b_hbm.at[sdidx[s].at[1]], bbuf[s],
                              bsem[s]).wait()
        pltpu.make_async_copy(ep_hbm.at[pl.ds(ebase, K)], epbuf[s],
                              esem[s]).wait()

    pltpu.sync_copy(sd_hbm.at[wrow], sdidx[0])
    pltpu.async_copy(sd_hbm.at[wrow + 1], sdidx[1], isem[1])
    issue_gathers(0, 0)

    def do_chunk(c, s):
        o = 1 - s

        @pl.when(c + 1 < NCHUNK)
        def _():
            pltpu.make_async_copy(sd_hbm.at[wrow], sdidx[o],
                                  isem[o]).wait()
            issue_gathers(c + 1, o)

        wait_gathers(s)

        hi_mask = jnp.full((L,), jnp.int32(-65536))
        epw = epbuf[s].bitcast(jnp.int32)

        def mrow(r2, c2):
            rlo = 2 * r2
            rhi = 2 * r2 + 1
            for c3 in range(D // L):
                sl = pl.ds(c3 * L, L)
                w = epw[r2, sl]
                elo = plsc.bitcast(w << 16, jnp.float32)
                ehi = plsc.bitcast(w & hi_mask, jnp.float32)
                mbuf[rlo, sl] = jnp.maximum(
                    abuf[s][rlo, sl] + bbuf[s][rlo, sl] + elo, 0.0)
                mbuf[rhi, sl] = jnp.maximum(
                    abuf[s][rhi, sl] + bbuf[s][rhi, sl] + ehi, 0.0)
            return c2
        lax.fori_loop(0, K // 2, mrow, 0)

        pltpu.sync_copy(mbuf, agg.at[sdidx[s].at[1]], add=True)

        @pl.when(c + 2 < NCHUNK)
        def _():
            pltpu.async_copy(sd_hbm.at[wrow + c + 2], sdidx[s], isem[s])

    def pair(g, carry):
        do_chunk(2 * g, 0)
        do_chunk(2 * g + 1, 1)
        return carry
    lax.fori_loop(0, NCHUNK // 2, pair, 0)

    plsc.subcore_barrier()

    def wcopy(p, carry):
        off = pl.multiple_of(sid * RPT + p * K, K)
        pltpu.sync_copy(agg.at[pl.ds(off, K)], mbuf)
        pltpu.sync_copy(mbuf, out_hbm.at[cid, pl.ds(off, K)])
        return carry
    lax.fori_loop(0, RPT // K, wcopy, 0)


@functools.partial(
    pl.kernel,
    out_type=jax.ShapeDtypeStruct((NC, NPAD, D), jnp.float32),
    mesh=plsc.VectorSubcoreMesh(core_axis_name="c", subcore_axis_name="s"),
    compiler_params=pltpu.CompilerParams(needs_layout_passes=False),
    scratch_types=[
        pltpu.VMEM((2, K), jnp.int32),
        pltpu.VMEM((2, K), jnp.int32),
        pltpu.VMEM((K, D), jnp.float32),
        pltpu.VMEM((K, D), jnp.float32),
        pltpu.VMEM((K, D), jnp.float32),
        pltpu.VMEM((K, D), jnp.float32),
        pltpu.VMEM((K, D), jnp.bfloat16),
        pltpu.VMEM((K, D), jnp.bfloat16),
        pltpu.VMEM((K, D), jnp.float32),
        pltpu.VMEM_SHARED((NPAD, D), jnp.float32),
        pltpu.SemaphoreType.DMA,
        pltpu.SemaphoreType.DMA,
        pltpu.SemaphoreType.DMA,
        pltpu.SemaphoreType.DMA,
        pltpu.SemaphoreType.DMA,
        pltpu.SemaphoreType.DMA,
        pltpu.SemaphoreType.DMA,
        pltpu.SemaphoreType.DMA,
    ],
)
def _sc_edge(a_hbm, b_hbm, ep_hbm, sd_hbm, out_hbm,
             sdidx0, sdidx1, a0, a1, b0, b1, e0, e1, mbuf, agg,
             isem0, isem1, asem0, asem1, bsem0, bsem1, esem0, esem1):
    _sc_edge_body(a_hbm, b_hbm, ep_hbm, sd_hbm, out_hbm,
                  (sdidx0, sdidx1), (a0, a1), (b0, b1), (e0, e1),
                  mbuf, agg, (isem0, isem1), (asem0, asem1), (bsem0, bsem1),
                  (esem0, esem1))


def kernel(node_features, edge_index, edge_features,
           enc_W1, enc_b1, enc_W2, enc_b2,
           msg_W0, msg_b0, upd_W0, upd_b0,
           msg_W1, msg_b1, upd_W1, upd_b1,
           msg_W2, msg_b2, upd_W2, upd_b2,
           dec_W1, dec_b1, dec_W2, dec_b2):
    sd = jnp.stack([edge_index[0].reshape(E // K, K),
                    edge_index[1].reshape(E // K, K)], axis=1)

    x = _mlp2(node_features, enc_W1, enc_b1, enc_W2, enc_b2, D)

    layers = [
        (msg_W0, msg_b0, upd_W0, upd_b0),
        (msg_W1, msg_b1, upd_W1, upd_b1),
        (msg_W2, msg_b2, upd_W2, upd_b2),
    ]
    for mW, mb, uW, ub in layers:
        a, b = _ab(x, mW[:D], mW[D:2 * D], mb)
        ep = _ep(edge_features, mW[2 * D:])
        aggs = _sc_edge(a, b, ep, sd)
        x = _upd(x, aggs, uW[:D], uW[D:], ub)

    return _mlp2(x, dec_W1, dec_b1, dec_W2, dec_b2, D)


# speedup vs baseline: 1.1648x; 1.1648x over previous
import functools

import jax
import jax.numpy as jnp
from jax import lax
from jax.experimental import pallas as pl
from jax.experimental.pallas import tpu as pltpu
from jax.experimental.pallas import tpu_sc as plsc

N = 10000
E = 320000
D = 128
ED = 16

NC = 2
NS = 16
L = 16
NW = NC * NS
EPW = E // NW
K = 40
NCHUNK = EPW // K
WROWS = NCHUNK
NPAD = 10240
RPT = NPAD // NS

NBLK = 1000
EBLK = 4000


def _mlp2_body(x_ref, w1_ref, b1_ref, w2_ref, b2_ref, o_ref):
    h = jnp.maximum(
        jnp.dot(x_ref[...], w1_ref[...], preferred_element_type=jnp.float32)
        + b1_ref[...], 0.0)
    o_ref[...] = (
        jnp.dot(h, w2_ref[...], preferred_element_type=jnp.float32)
        + b2_ref[...])


def _mlp2(x, w1, b1, w2, b2, dout):
    n = x.shape[0]
    return pl.pallas_call(
        _mlp2_body,
        grid=(n // NBLK,),
        in_specs=[
            pl.BlockSpec((NBLK, x.shape[1]), lambda i: (i, 0)),
            pl.BlockSpec(w1.shape, lambda i: (0, 0)),
            pl.BlockSpec((1, D), lambda i: (0, 0)),
            pl.BlockSpec(w2.shape, lambda i: (0, 0)),
            pl.BlockSpec((1, dout), lambda i: (0, 0)),
        ],
        out_specs=pl.BlockSpec((NBLK, dout), lambda i: (i, 0)),
        out_shape=jax.ShapeDtypeStruct((n, dout), jnp.float32),
    )(x, w1, b1.reshape(1, D), w2, b2.reshape(1, dout))


def _ab_body(x_ref, wa_ref, wb_ref, mb_ref, a_ref, b_ref):
    xv = x_ref[...]
    a_ref[...] = jnp.dot(xv, wa_ref[...], preferred_element_type=jnp.float32)
    b_ref[...] = (
        jnp.dot(xv, wb_ref[...], preferred_element_type=jnp.float32)
        + mb_ref[...])


def _ab(x, wa, wb, mb):
    return pl.pallas_call(
        _ab_body,
        grid=(N // NBLK,),
        in_specs=[
            pl.BlockSpec((NBLK, D), lambda i: (i, 0)),
            pl.BlockSpec((D, D), lambda i: (0, 0)),
            pl.BlockSpec((D, D), lambda i: (0, 0)),
            pl.BlockSpec((1, D), lambda i: (0, 0)),
        ],
        out_specs=[
            pl.BlockSpec((NBLK, D), lambda i: (i, 0)),
            pl.BlockSpec((NBLK, D), lambda i: (i, 0)),
        ],
        out_shape=[
            jax.ShapeDtypeStruct((N, D), jnp.float32),
            jax.ShapeDtypeStruct((N, D), jnp.float32),
        ],
    )(x, wa, wb, mb.reshape(1, D))


def _ep_body(ef_ref, we_ref, o_ref):
    o_ref[...] = jnp.dot(ef_ref[...], we_ref[...],
                         preferred_element_type=jnp.float32)


def _ep(ef, we):
    return pl.pallas_call(
        _ep_body,
        grid=(E // EBLK,),
        in_specs=[
            pl.BlockSpec((EBLK, ED), lambda i: (i, 0)),
            pl.BlockSpec((ED, D), lambda i: (0, 0)),
        ],
        out_specs=pl.BlockSpec((EBLK, D), lambda i: (i, 0)),
        out_shape=jax.ShapeDtypeStruct((E, D), jnp.float32),
    )(ef, we)


def _upd_body(x_ref, g0_ref, g1_ref, wx_ref, wa_ref, ub_ref, o_ref):
    xv = x_ref[...]
    agg = g0_ref[0] + g1_ref[0]
    o_ref[...] = xv + jnp.maximum(
        jnp.dot(xv, wx_ref[...], preferred_element_type=jnp.float32)
        + jnp.dot(agg, wa_ref[...], preferred_element_type=jnp.float32)
        + ub_ref[...], 0.0)


def _upd(x, aggs, wx, wa, ub):
    return pl.pallas_call(
        _upd_body,
        grid=(N // NBLK,),
        in_specs=[
            pl.BlockSpec((NBLK, D), lambda i: (i, 0)),
            pl.BlockSpec((1, NBLK, D), lambda i: (0, i, 0)),
            pl.BlockSpec((1, NBLK, D), lambda i: (1, i, 0)),
            pl.BlockSpec((D, D), lambda i: (0, 0)),
            pl.BlockSpec((D, D), lambda i: (0, 0)),
            pl.BlockSpec((1, D), lambda i: (0, 0)),
        ],
        out_specs=pl.BlockSpec((NBLK, D), lambda i: (i, 0)),
        out_shape=jax.ShapeDtypeStruct((N, D), jnp.float32),
    )(x, aggs, aggs, wx, wa, ub.reshape(1, D))


def _sc_edge_body(a_hbm, b_hbm, ep_hbm, sd_hbm, out_hbm,
                  sdidx, abuf, bbuf, epbuf, agg, isem, asem, bsem,
                  esem):
    cid = lax.axis_index("c")
    sid = lax.axis_index("s")
    wid = sid * NC + cid
    wrow = wid * WROWS
    ebase = wid * EPW

    def zrow(r, carry):
        for c in range(D // L):
            abuf[0][r, pl.ds(c * L, L)] = jnp.zeros((L,), jnp.float32)
        return carry
    lax.fori_loop(0, K, zrow, 0)

    def zcopy(p, carry):
        off = pl.multiple_of(sid * RPT + p * K, K)
        pltpu.sync_copy(abuf[0], agg.at[pl.ds(off, K)])
        return carry
    lax.fori_loop(0, RPT // K, zcopy, 0)
    plsc.subcore_barrier()

    def issue_gathers(c, s):
        pltpu.async_copy(a_hbm.at[sdidx[s].at[0]], abuf[s], asem[s])
        pltpu.async_copy(b_hbm.at[sdidx[s].at[1]], bbuf[s], bsem[s])
        pltpu.async_copy(ep_hbm.at[pl.ds(ebase + c * K, K)], epbuf[s],
                         esem[s])

    def wait_gathers(s):
        pltpu.make_async_copy(a_hbm.at[sdidx[s].at[0]], abuf[s],
                              asem[s]).wait()
        pltpu.make_async_copy(b_hbm.at[sdidx[s].at[1]], bbuf[s],
                              bsem[s]).wait()
        pltpu.make_async_copy(ep_hbm.at[pl.ds(ebase, K)], epbuf[s],
                              esem[s]).wait()

    pltpu.sync_copy(sd_hbm.at[wrow], sdidx[0])
    pltpu.async_copy(sd_hbm.at[wrow + 1], sdidx[1], isem[1])
    issue_gathers(0, 0)

    def do_chunk(c, s):
        o = 1 - s

        @pl.when(c + 1 < NCHUNK)
        def _():
            pltpu.make_async_copy(sd_hbm.at[wrow], sdidx[o],
                                  isem[o]).wait()
            issue_gathers(c + 1, o)

        wait_gathers(s)

        def mrow(r, c2):
            for c3 in range(D // L):
                sl = pl.ds(c3 * L, L)
                abuf[s][r, sl] = jnp.maximum(
                    abuf[s][r, sl] + bbuf[s][r, sl] + epbuf[s][r, sl], 0.0)
            return c2
        lax.fori_loop(0, K, mrow, 0)

        pltpu.sync_copy(abuf[s], agg.at[sdidx[s].at[1]], add=True)

        @pl.when(c + 2 < NCHUNK)
        def _():
            pltpu.async_copy(sd_hbm.at[wrow + c + 2], sdidx[s], isem[s])

    def pair(g, carry):
        do_chunk(2 * g, 0)
        do_chunk(2 * g + 1, 1)
        return carry
    lax.fori_loop(0, NCHUNK // 2, pair, 0)

    plsc.subcore_barrier()

    def wcopy(p, carry):
        off = pl.multiple_of(sid * RPT + p * K, K)
        pltpu.sync_copy(agg.at[pl.ds(off, K)], abuf[0])
        pltpu.sync_copy(abuf[0], out_hbm.at[cid, pl.ds(off, K)])
        return carry
    lax.fori_loop(0, RPT // K, wcopy, 0)


@functools.partial(
    pl.kernel,
    out_type=jax.ShapeDtypeStruct((NC, NPAD, D), jnp.float32),
    mesh=plsc.VectorSubcoreMesh(core_axis_name="c", subcore_axis_name="s"),
    scratch_types=[
        pltpu.VMEM((2, K), jnp.int32),
        pltpu.VMEM((2, K), jnp.int32),
        pltpu.VMEM((K, D), jnp.float32),
        pltpu.VMEM((K, D), jnp.float32),
        pltpu.VMEM((K, D), jnp.float32),
        pltpu.VMEM((K, D), jnp.float32),
        pltpu.VMEM((K, D), jnp.float32),
        pltpu.VMEM((K, D), jnp.float32),
        pltpu.VMEM_SHARED((NPAD, D), jnp.float32),
        pltpu.SemaphoreType.DMA,
        pltpu.SemaphoreType.DMA,
        pltpu.SemaphoreType.DMA,
        pltpu.SemaphoreType.DMA,
        pltpu.SemaphoreType.DMA,
        pltpu.SemaphoreType.DMA,
        pltpu.SemaphoreType.DMA,
        pltpu.SemaphoreType.DMA,
    ],
)
def _sc_edge(a_hbm, b_hbm, ep_hbm, sd_hbm, out_hbm,
             sdidx0, sdidx1, a0, a1, b0, b1, e0, e1, agg,
             isem0, isem1, asem0, asem1, bsem0, bsem1, esem0, esem1):
    _sc_edge_body(a_hbm, b_hbm, ep_hbm, sd_hbm, out_hbm,
                  (sdidx0, sdidx1), (a0, a1), (b0, b1), (e0, e1),
                  agg, (isem0, isem1), (asem0, asem1), (bsem0, bsem1),
                  (esem0, esem1))


def kernel(node_features, edge_index, edge_features,
           enc_W1, enc_b1, enc_W2, enc_b2,
           msg_W0, msg_b0, upd_W0, upd_b0,
           msg_W1, msg_b1, upd_W1, upd_b1,
           msg_W2, msg_b2, upd_W2, upd_b2,
           dec_W1, dec_b1, dec_W2, dec_b2):
    sd = jnp.stack([edge_index[0].reshape(E // K, K),
                    edge_index[1].reshape(E // K, K)], axis=1)

    x = _mlp2(node_features, enc_W1, enc_b1, enc_W2, enc_b2, D)

    layers = [
        (msg_W0, msg_b0, upd_W0, upd_b0),
        (msg_W1, msg_b1, upd_W1, upd_b1),
        (msg_W2, msg_b2, upd_W2, upd_b2),
    ]
    eps = [_ep(edge_features, mW[2 * D:]) for mW, _, _, _ in layers]

    for (mW, mb, uW, ub), ep in zip(layers, eps):
        a, b = _ab(x, mW[:D], mW[D:2 * D], mb)
        aggs = _sc_edge(a, b, ep, sd)
        x = _upd(x, aggs, uW[:D], uW[D:], ub)

    return _mlp2(x, dec_W1, dec_b1, dec_W2, dec_b2, D)


# speedup vs baseline: 1.1934x; 1.0246x over previous
import functools

import jax
import jax.numpy as jnp
from jax import lax
from jax.experimental import pallas as pl
from jax.experimental.pallas import tpu as pltpu
from jax.experimental.pallas import tpu_sc as plsc

N = 10000
E = 320000
D = 128
ED = 16

NC = 2
NS = 16
L = 16
NW = NC * NS
EPW = E // NW
K = 40
NCHUNK = EPW // K
WROWS = NCHUNK
NPAD = 10240
RPT = NPAD // NS

NBLK = 1000
EBLK = 4000


def _encab_body(nf_ref, w1_ref, b1_ref, w2_ref, b2_ref,
                wa_ref, wb_ref, mb_ref, x_ref, a_ref, b_ref):
    h = jnp.maximum(
        jnp.dot(nf_ref[...], w1_ref[...], preferred_element_type=jnp.float32)
        + b1_ref[...], 0.0)
    x = (jnp.dot(h, w2_ref[...], preferred_element_type=jnp.float32)
         + b2_ref[...])
    x_ref[...] = x
    a_ref[...] = jnp.dot(x, wa_ref[...], preferred_element_type=jnp.float32)
    b_ref[...] = (
        jnp.dot(x, wb_ref[...], preferred_element_type=jnp.float32)
        + mb_ref[...])


def _encab(nf, w1, b1, w2, b2, wa, wb, mb):
    fin = nf.shape[1]
    return pl.pallas_call(
        _encab_body,
        grid=(N // NBLK,),
        in_specs=[
            pl.BlockSpec((NBLK, fin), lambda i: (i, 0)),
            pl.BlockSpec(w1.shape, lambda i: (0, 0)),
            pl.BlockSpec((1, D), lambda i: (0, 0)),
            pl.BlockSpec(w2.shape, lambda i: (0, 0)),
            pl.BlockSpec((1, D), lambda i: (0, 0)),
            pl.BlockSpec((D, D), lambda i: (0, 0)),
            pl.BlockSpec((D, D), lambda i: (0, 0)),
            pl.BlockSpec((1, D), lambda i: (0, 0)),
        ],
        out_specs=[
            pl.BlockSpec((NBLK, D), lambda i: (i, 0)),
            pl.BlockSpec((NBLK, D), lambda i: (i, 0)),
            pl.BlockSpec((NBLK, D), lambda i: (i, 0)),
        ],
        out_shape=[
            jax.ShapeDtypeStruct((N, D), jnp.float32),
            jax.ShapeDtypeStruct((N, D), jnp.float32),
            jax.ShapeDtypeStruct((N, D), jnp.float32),
        ],
    )(nf, w1, b1.reshape(1, D), w2, b2.reshape(1, D),
      wa, wb, mb.reshape(1, D))


def _ep3_body(ef_ref, w0_ref, w1_ref, w2_ref, o0_ref, o1_ref, o2_ref):
    efv = ef_ref[...]
    o0_ref[...] = jnp.dot(efv, w0_ref[...], preferred_element_type=jnp.float32)
    o1_ref[...] = jnp.dot(efv, w1_ref[...], preferred_element_type=jnp.float32)
    o2_ref[...] = jnp.dot(efv, w2_ref[...], preferred_element_type=jnp.float32)


def _ep3(ef, w0, w1, w2):
    wspec = pl.BlockSpec((ED, D), lambda i: (0, 0))
    ospec = pl.BlockSpec((EBLK, D), lambda i: (i, 0))
    oshape = jax.ShapeDtypeStruct((E, D), jnp.float32)
    return pl.pallas_call(
        _ep3_body,
        grid=(E // EBLK,),
        in_specs=[pl.BlockSpec((EBLK, ED), lambda i: (i, 0)),
                  wspec, wspec, wspec],
        out_specs=[ospec, ospec, ospec],
        out_shape=[oshape, oshape, oshape],
    )(ef, w0, w1, w2)


def _updab_body(x_ref, g0_ref, g1_ref, wx_ref, wa_ref, ub_ref,
                nwa_ref, nwb_ref, nmb_ref, xo_ref, ao_ref, bo_ref):
    xv = x_ref[...]
    agg = g0_ref[0] + g1_ref[0]
    xn = xv + jnp.maximum(
        jnp.dot(xv, wx_ref[...], preferred_element_type=jnp.float32)
        + jnp.dot(agg, wa_ref[...], preferred_element_type=jnp.float32)
        + ub_ref[...], 0.0)
    xo_ref[...] = xn
    ao_ref[...] = jnp.dot(xn, nwa_ref[...], preferred_element_type=jnp.float32)
    bo_ref[...] = (
        jnp.dot(xn, nwb_ref[...], preferred_element_type=jnp.float32)
        + nmb_ref[...])


def _updab(x, aggs, wx, wa, ub, nwa, nwb, nmb):
    dspec = pl.BlockSpec((D, D), lambda i: (0, 0))
    bspec = pl.BlockSpec((1, D), lambda i: (0, 0))
    nspec = pl.BlockSpec((NBLK, D), lambda i: (i, 0))
    nshape = jax.ShapeDtypeStruct((N, D), jnp.float32)
    return pl.pallas_call(
        _updab_body,
        grid=(N // NBLK,),
        in_specs=[
            nspec,
            pl.BlockSpec((1, NBLK, D), lambda i: (0, i, 0)),
            pl.BlockSpec((1, NBLK, D), lambda i: (1, i, 0)),
            dspec, dspec, bspec, dspec, dspec, bspec,
        ],
        out_specs=[nspec, nspec, nspec],
        out_shape=[nshape, nshape, nshape],
    )(x, aggs, aggs, wx, wa, ub.reshape(1, D),
      nwa, nwb, nmb.reshape(1, D))


def _upddec_body(x_ref, g0_ref, g1_ref, wx_ref, wa_ref, ub_ref,
                 dw1_ref, db1_ref, dw2_ref, db2_ref, o_ref):
    xv = x_ref[...]
    agg = g0_ref[0] + g1_ref[0]
    xn = xv + jnp.maximum(
        jnp.dot(xv, wx_ref[...], preferred_element_type=jnp.float32)
        + jnp.dot(agg, wa_ref[...], preferred_element_type=jnp.float32)
        + ub_ref[...], 0.0)
    h = jnp.maximum(
        jnp.dot(xn, dw1_ref[...], preferred_element_type=jnp.float32)
        + db1_ref[...], 0.0)
    o_ref[...] = (
        jnp.dot(h, dw2_ref[...], preferred_element_type=jnp.float32)
        + db2_ref[...])


def _upddec(x, aggs, wx, wa, ub, dw1, db1, dw2, db2):
    dout = dw2.shape[1]
    dspec = pl.BlockSpec((D, D), lambda i: (0, 0))
    bspec = pl.BlockSpec((1, D), lambda i: (0, 0))
    return pl.pallas_call(
        _upddec_body,
        grid=(N // NBLK,),
        in_specs=[
            pl.BlockSpec((NBLK, D), lambda i: (i, 0)),
            pl.BlockSpec((1, NBLK, D), lambda i: (0, i, 0)),
            pl.BlockSpec((1, NBLK, D), lambda i: (1, i, 0)),
            dspec, dspec, bspec,
            pl.BlockSpec(dw1.shape, lambda i: (0, 0)),
            bspec,
            pl.BlockSpec(dw2.shape, lambda i: (0, 0)),
            pl.BlockSpec((1, dout), lambda i: (0, 0)),
        ],
        out_specs=pl.BlockSpec((NBLK, dout), lambda i: (i, 0)),
        out_shape=jax.ShapeDtypeStruct((N, dout), jnp.float32),
    )(x, aggs, aggs, wx, wa, ub.reshape(1, D),
      dw1, db1.reshape(1, D), dw2, db2.reshape(1, dout))


def _sc_edge_body(a_hbm, b_hbm, ep_hbm, sd_hbm, out_hbm,
                  sdidx, abuf, bbuf, epbuf, agg, isem, asem, bsem,
                  esem):
    cid = lax.axis_index("c")
    sid = lax.axis_index("s")
    wid = sid * NC + cid
    wrow = wid * WROWS
    ebase = wid * EPW

    def zrow(r, carry):
        for c in range(D // L):
            abuf[0][r, pl.ds(c * L, L)] = jnp.zeros((L,), jnp.float32)
        return carry
    lax.fori_loop(0, K, zrow, 0)

    def zcopy(p, carry):
        off = pl.multiple_of(sid * RPT + p * K, K)
        pltpu.sync_copy(abuf[0], agg.at[pl.ds(off, K)])
        return carry
    lax.fori_loop(0, RPT // K, zcopy, 0)
    plsc.subcore_barrier()

    def issue_gathers(c, s):
        pltpu.async_copy(a_hbm.at[sdidx[s].at[0]], abuf[s], asem[s])
        pltpu.async_copy(b_hbm.at[sdidx[s].at[1]], bbuf[s], bsem[s])
        pltpu.async_copy(ep_hbm.at[pl.ds(ebase + c * K, K)], epbuf[s],
                         esem[s])

    def wait_gathers(s):
        pltpu.make_async_copy(a_hbm.at[sdidx[s].at[0]], abuf[s],
                              asem[s]).wait()
        pltpu.make_async_copy(b_hbm.at[sdidx[s].at[1]], bbuf[s],
                              bsem[s]).wait()
        pltpu.make_async_copy(ep_hbm.at[pl.ds(ebase, K)], epbuf[s],
                              esem[s]).wait()

    pltpu.sync_copy(sd_hbm.at[wrow], sdidx[0])
    pltpu.async_copy(sd_hbm.at[wrow + 1], sdidx[1], isem[1])
    issue_gathers(0, 0)

    def do_chunk(c, s):
        o = 1 - s

        @pl.when(c + 1 < NCHUNK)
        def _():
            pltpu.make_async_copy(sd_hbm.at[wrow], sdidx[o],
                                  isem[o]).wait()
            issue_gathers(c + 1, o)

        wait_gathers(s)

        def mrow(r, c2):
            for c3 in range(D // L):
                sl = pl.ds(c3 * L, L)
                abuf[s][r, sl] = jnp.maximum(
                    abuf[s][r, sl] + bbuf[s][r, sl] + epbuf[s][r, sl], 0.0)
            return c2
        lax.fori_loop(0, K, mrow, 0)

        pltpu.sync_copy(abuf[s], agg.at[sdidx[s].at[1]], add=True)

        @pl.when(c + 2 < NCHUNK)
        def _():
            pltpu.async_copy(sd_hbm.at[wrow + c + 2], sdidx[s], isem[s])

    def pair(g, carry):
        do_chunk(2 * g, 0)
        do_chunk(2 * g + 1, 1)
        return carry
    lax.fori_loop(0, NCHUNK // 2, pair, 0)

    plsc.subcore_barrier()

    def wcopy(p, carry):
        off = pl.multiple_of(sid * RPT + p * K, K)
        pltpu.sync_copy(agg.at[pl.ds(off, K)], abuf[0])
        pltpu.sync_copy(abuf[0], out_hbm.at[cid, pl.ds(off, K)])
        return carry
    lax.fori_loop(0, RPT // K, wcopy, 0)


@functools.partial(
    pl.kernel,
    out_type=jax.ShapeDtypeStruct((NC, NPAD, D), jnp.float32),
    mesh=plsc.VectorSubcoreMesh(core_axis_name="c", subcore_axis_name="s"),
    scratch_types=[
        pltpu.VMEM((2, K), jnp.int32),
        pltpu.VMEM((2, K), jnp.int32),
        pltpu.VMEM((K, D), jnp.float32),
        pltpu.VMEM((K, D), jnp.float32),
        pltpu.VMEM((K, D), jnp.float32),
        pltpu.VMEM((K, D), jnp.float32),
        pltpu.VMEM((K, D), jnp.float32),
        pltpu.VMEM((K, D), jnp.float32),
        pltpu.VMEM_SHARED((NPAD, D), jnp.float32),
        pltpu.SemaphoreType.DMA,
        pltpu.SemaphoreType.DMA,
        pltpu.SemaphoreType.DMA,
        pltpu.SemaphoreType.DMA,
        pltpu.SemaphoreType.DMA,
        pltpu.SemaphoreType.DMA,
        pltpu.SemaphoreType.DMA,
        pltpu.SemaphoreType.DMA,
    ],
)
def _sc_edge(a_hbm, b_hbm, ep_hbm, sd_hbm, out_hbm,
             sdidx0, sdidx1, a0, a1, b0, b1, e0, e1, agg,
             isem0, isem1, asem0, asem1, bsem0, bsem1, esem0, esem1):
    _sc_edge_body(a_hbm, b_hbm, ep_hbm, sd_hbm, out_hbm,
                  (sdidx0, sdidx1), (a0, a1), (b0, b1), (e0, e1),
                  agg, (isem0, isem1), (asem0, asem1), (bsem0, bsem1),
                  (esem0, esem1))


def kernel(node_features, edge_index, edge_features,
           enc_W1, enc_b1, enc_W2, enc_b2,
           msg_W0, msg_b0, upd_W0, upd_b0,
           msg_W1, msg_b1, upd_W1, upd_b1,
           msg_W2, msg_b2, upd_W2, upd_b2,
           dec_W1, dec_b1, dec_W2, dec_b2):
    sd = jnp.stack([edge_index[0].reshape(E // K, K),
                    edge_index[1].reshape(E // K, K)], axis=1)

    ep0, ep1, ep2 = _ep3(edge_features,
                         msg_W0[2 * D:], msg_W1[2 * D:], msg_W2[2 * D:])

    x, a, b = _encab(node_features, enc_W1, enc_b1, enc_W2, enc_b2,
                     msg_W0[:D], msg_W0[D:2 * D], msg_b0)
    aggs = _sc_edge(a, b, ep0, sd)
    x, a, b = _updab(x, aggs, upd_W0[:D], upd_W0[D:], upd_b0,
                     msg_W1[:D], msg_W1[D:2 * D], msg_b1)
    aggs = _sc_edge(a, b, ep1, sd)
    x, a, b = _updab(x, aggs, upd_W1[:D], upd_W1[D:], upd_b1,
                     msg_W2[:D], msg_W2[D:2 * D], msg_b2)
    aggs = _sc_edge(a, b, ep2, sd)
    return _upddec(x, aggs, upd_W2[:D], upd_W2[D:], upd_b2,
                   dec_W1, dec_b1, dec_W2, dec_b2)
